# Initial kernel scaffold; baseline (speedup 1.0000x reference)
import jax
import jax.numpy as jnp
from jax.experimental import pallas as pl


def kernel(x, edge_index, W1, b1, bn_gamma, bn_beta, W2, b2):
    raise NotImplementedError("write your pallas kernel here")


# speedup vs baseline: 14.5071x; 14.5071x over previous
import functools

import jax
import jax.numpy as jnp
from jax import lax
from jax.experimental import pallas as pl
from jax.experimental.pallas import tpu as pltpu
from jax.experimental.pallas import tpu_sc as plsc

N = 10000
E = 320000
D = 128
NC = 2
NS = 16
NW = NC * NS
NPAD = 10240
NPT = NPAD // NS
B = 128
CH = 79
EPW = CH * B
EPAD = NW * EPW

_mesh = plsc.VectorSubcoreMesh(core_axis_name="c", subcore_axis_name="s")


@functools.partial(
    pl.kernel,
    out_type=jax.ShapeDtypeStruct((NC * NPAD,), jnp.float32),
    mesh=_mesh,
    scratch_types=[
        pltpu.VMEM((CH, B), jnp.int32),
        pltpu.VMEM((B,), jnp.float32),
        pltpu.VMEM((NPT,), jnp.float32),
        pltpu.VMEM_SHARED((NPAD,), jnp.float32),
    ],
)
def _deg_sc(dst_hbm, out_hbm, didx, ones_v, zb, deg_sh):
    c = lax.axis_index("c")
    s = lax.axis_index("s")
    w = c * NS + s
    for i in range(NPT // 16):
        zb[pl.ds(i * 16, 16)] = jnp.zeros((16,), jnp.float32)
    for i in range(B // 16):
        ones_v[pl.ds(i * 16, 16)] = jnp.ones((16,), jnp.float32)
    pltpu.sync_copy(zb, deg_sh.at[pl.ds(s * NPT, NPT)])
    pltpu.sync_copy(dst_hbm.at[w], didx)
    plsc.subcore_barrier()

    def chunk(j, carry):
        pltpu.sync_copy(ones_v, deg_sh.at[didx.at[j]], add=True)
        return carry

    lax.fori_loop(0, CH, chunk, 0)
    plsc.subcore_barrier()
    pltpu.sync_copy(deg_sh.at[pl.ds(s * NPT, NPT)],
                    out_hbm.at[pl.ds(c * NPAD + s * NPT, NPT)])


@functools.partial(
    pl.kernel,
    out_type=jax.ShapeDtypeStruct((NC, NPAD, D), jnp.float32),
    mesh=_mesh,
    scratch_types=[
        pltpu.VMEM((CH, B), jnp.int32),
        pltpu.VMEM((CH, B), jnp.int32),
        pltpu.VMEM((B, D), jnp.float32),
        pltpu.SemaphoreType.DMA,
        pltpu.VMEM_SHARED((NPAD, D), jnp.float32),
    ],
)
def _scatter_sc(h_hbm, src_hbm, dst_hbm, out_hbm, sidx, didx, rows, sem, acc):
    c = lax.axis_index("c")
    s = lax.axis_index("s")
    w = c * NS + s
    pltpu.sync_copy(src_hbm.at[w], sidx)
    pltpu.sync_copy(dst_hbm.at[w], didx)

    @pl.when(c == 0)
    def _():
        pltpu.sync_copy(h_hbm.at[pl.ds(s * NPT, NPT)],
                        acc.at[pl.ds(s * NPT, NPT)])

    @pl.when(c == 1)
    def _():
        def zrow(t, carry):
            rows[t // 8, pl.ds((t % 8) * 16, 16)] = jnp.zeros((16,), jnp.float32)
            return carry
        lax.fori_loop(0, B * (D // 16), zrow, 0)
        for k in range(NPT // B):
            pltpu.sync_copy(rows, acc.at[pl.ds(s * NPT + k * B, B)])

    plsc.subcore_barrier()

    def chunk(j, carry):
        pltpu.async_copy(h_hbm.at[sidx.at[j]], rows, sem).wait()
        pltpu.sync_copy(rows, acc.at[didx.at[j]], add=True)
        return carry

    lax.fori_loop(0, CH, chunk, 0)
    plsc.subcore_barrier()
    pltpu.sync_copy(acc.at[pl.ds(s * NPT, NPT)],
                    out_hbm.at[c, pl.ds(s * NPT, NPT)])


def _prep_body(deg_ref, x_ref, w1_ref, h_ref, dinv_ref):
    deg = deg_ref[pl.ds(0, NPAD)] + deg_ref[pl.ds(NPAD, NPAD)] + 1.0
    dinv = lax.rsqrt(deg).reshape(NPAD, 1)
    dinv_ref[...] = dinv
    h = jnp.dot(x_ref[...], w1_ref[...], preferred_element_type=jnp.float32)
    h = h * dinv[:N]
    h_ref[...] = jnp.concatenate(
        [h, jnp.zeros((NPAD - N, D), jnp.float32)], axis=0)


def _mid_body(p_ref, dinv_ref, b1_ref, g_ref, be_ref, w2_ref, out_ref):
    dinv = dinv_ref[pl.ds(0, N)]
    z = (p_ref[0, pl.ds(0, N)] + p_ref[1, pl.ds(0, N)]) * dinv \
        + b1_ref[...].reshape(1, D)
    mean = jnp.mean(z, axis=0, keepdims=True)
    cz = z - mean
    var = jnp.mean(cz * cz, axis=0, keepdims=True)
    y = cz * lax.rsqrt(var + 1e-5) * g_ref[...].reshape(1, D) \
        + be_ref[...].reshape(1, D)
    y = jnp.maximum(y, 0.0)
    h2 = jnp.dot(y, w2_ref[...], preferred_element_type=jnp.float32) * dinv
    out_ref[...] = jnp.concatenate(
        [h2, jnp.zeros((NPAD - N, D), jnp.float32)], axis=0)


def _final_body(p_ref, dinv_ref, b2_ref, out_ref):
    out_ref[...] = (p_ref[0, pl.ds(0, N)] + p_ref[1, pl.ds(0, N)]) \
        * dinv_ref[pl.ds(0, N)] + b2_ref[...].reshape(1, D)


_prep_tc = pl.pallas_call(
    _prep_body,
    out_shape=[jax.ShapeDtypeStruct((NPAD, D), jnp.float32),
               jax.ShapeDtypeStruct((NPAD, 1), jnp.float32)],
)
_mid_tc = pl.pallas_call(
    _mid_body,
    out_shape=jax.ShapeDtypeStruct((NPAD, D), jnp.float32),
)
_final_tc = pl.pallas_call(
    _final_body,
    out_shape=jax.ShapeDtypeStruct((N, D), jnp.float32),
)


def kernel(x, edge_index, W1, b1, bn_gamma, bn_beta, W2, b2):
    src = edge_index[0].astype(jnp.int32)
    dst = edge_index[1].astype(jnp.int32)
    pad = EPAD - E
    fill = jnp.full((pad,), N, jnp.int32)
    srcp = jnp.concatenate([src, fill]).reshape(NW, CH, B)
    dstp = jnp.concatenate([dst, fill]).reshape(NW, CH, B)

    degp = _deg_sc(dstp)
    h1, dinv = _prep_tc(degp, x, W1)
    p1 = _scatter_sc(h1, srcp, dstp)
    h2 = _mid_tc(p1, dinv, b1, bn_gamma, bn_beta, W2)
    p2 = _scatter_sc(h2, srcp, dstp)
    return _final_tc(p2, dinv, b2)
